# Initial kernel scaffold; baseline (speedup 1.0000x reference)
import jax
import jax.numpy as jnp
from jax.experimental import pallas as pl


def kernel(nodes, edges, W1, b1, W2, b2, W3, b3):
    raise NotImplementedError("write your pallas kernel here")


# speedup vs baseline: 40.0702x; 40.0702x over previous
import functools

import jax
import jax.numpy as jnp
from jax import lax
from jax.experimental import pallas as pl
from jax.experimental.pallas import tpu as pltpu
import jax.experimental.pallas.tpu_sc as plsc

N_NODES = 50000
N_EDGES = 1600000
NC = 2
NS = 16
NW = NC * NS
N_PAD = 50048
R_TILE = N_PAD // NS
CHUNK = 2000
PER_W = N_EDGES // NW

_mesh = plsc.VectorSubcoreMesh(core_axis_name="c", subcore_axis_name="s")
_sc_params = pltpu.CompilerParams(use_tc_tiling_on_sc=False)


@functools.partial(
    pl.kernel,
    out_type=jax.ShapeDtypeStruct((NC, N_PAD, 1), jnp.float32),
    mesh=_mesh,
    compiler_params=_sc_params,
    scratch_types=[
        pltpu.VMEM((CHUNK,), jnp.int32),
        pltpu.VMEM((CHUNK, 1), jnp.float32),
        pltpu.VMEM((R_TILE, 1), jnp.float32),
        pltpu.VMEM_SHARED((N_PAD, 1), jnp.float32),
        pltpu.SemaphoreType.DMA,
    ],
)
def _sc_count(dst_hbm, ones_hbm, zeros_hbm, out_hbm,
              dst_v, ones_v, buf_v, acc_sh, sem):
    cid = lax.axis_index("c")
    sid = lax.axis_index("s")
    wid = cid * NS + sid
    pltpu.sync_copy(zeros_hbm.at[pl.ds(sid * R_TILE, R_TILE)], buf_v)
    pltpu.sync_copy(buf_v, acc_sh.at[pl.ds(sid * R_TILE, R_TILE)])
    pltpu.sync_copy(ones_hbm, ones_v)
    plsc.subcore_barrier()
    base = wid * PER_W

    def body(i, carry):
        off = base + i * CHUNK
        pltpu.sync_copy(dst_hbm.at[pl.ds(off, CHUNK)], dst_v)
        pltpu.sync_copy(ones_v, acc_sh.at[dst_v], add=True)
        return carry

    lax.fori_loop(0, PER_W // CHUNK, body, 0)
    plsc.subcore_barrier()
    pltpu.sync_copy(acc_sh.at[pl.ds(sid * R_TILE, R_TILE)], buf_v)
    pltpu.sync_copy(buf_v, out_hbm.at[cid, pl.ds(sid * R_TILE, R_TILE)])


@functools.partial(
    pl.kernel,
    out_type=jax.ShapeDtypeStruct((NC, N_PAD, 3), jnp.float32),
    mesh=_mesh,
    compiler_params=_sc_params,
    scratch_types=[
        pltpu.VMEM((CHUNK,), jnp.int32),
        pltpu.VMEM((CHUNK,), jnp.int32),
        pltpu.VMEM((CHUNK, 3), jnp.float32),
        pltpu.VMEM((R_TILE, 3), jnp.float32),
        pltpu.VMEM_SHARED((N_PAD, 3), jnp.float32),
        pltpu.SemaphoreType.DMA,
    ],
)
def _sc_scatter(g_hbm, src_hbm, dst_hbm, zeros_hbm, out_hbm,
                src_v, dst_v, rows_v, buf_v, acc_sh, sem):
    cid = lax.axis_index("c")
    sid = lax.axis_index("s")
    wid = cid * NS + sid
    pltpu.sync_copy(zeros_hbm.at[pl.ds(sid * R_TILE, R_TILE)], buf_v)
    pltpu.sync_copy(buf_v, acc_sh.at[pl.ds(sid * R_TILE, R_TILE)])
    plsc.subcore_barrier()
    base = wid * PER_W

    def body(i, carry):
        off = base + i * CHUNK
        pltpu.sync_copy(src_hbm.at[pl.ds(off, CHUNK)], src_v)
        pltpu.async_copy(g_hbm.at[src_v], rows_v, sem).wait()
        pltpu.sync_copy(dst_hbm.at[pl.ds(off, CHUNK)], dst_v)
        pltpu.sync_copy(rows_v, acc_sh.at[dst_v], add=True)
        return carry

    lax.fori_loop(0, PER_W // CHUNK, body, 0)
    plsc.subcore_barrier()
    pltpu.sync_copy(acc_sh.at[pl.ds(sid * R_TILE, R_TILE)], buf_v)
    pltpu.sync_copy(buf_v, out_hbm.at[cid, pl.ds(sid * R_TILE, R_TILE)])


_B = 1000
_GRID = N_NODES // _B


def _tc_prep_body(cnt_ref, nodes_ref, w1_ref, g1_ref, dis_ref):
    deg = cnt_ref[:, 0] + cnt_ref[:, 1] + 1.0
    dis = lax.rsqrt(deg)[:, None]
    h = lax.dot_general(nodes_ref[...], w1_ref[...],
                        (((1,), (1,)), ((), ())),
                        preferred_element_type=jnp.float32)
    g1_ref[...] = h * dis
    dis_ref[...] = dis


_tc_prep = pl.pallas_call(
    _tc_prep_body,
    grid=(_GRID,),
    in_specs=[
        pl.BlockSpec((_B, 2), lambda i: (i, 0)),
        pl.BlockSpec((_B, 3), lambda i: (i, 0)),
        pl.BlockSpec((3, 3), lambda i: (0, 0)),
    ],
    out_specs=[
        pl.BlockSpec((_B, 3), lambda i: (i, 0)),
        pl.BlockSpec((_B, 1), lambda i: (i, 0)),
    ],
    out_shape=[
        jax.ShapeDtypeStruct((N_NODES, 3), jnp.float32),
        jax.ShapeDtypeStruct((N_NODES, 1), jnp.float32),
    ],
)


def _tc_mid_body(s1a_ref, s1b_ref, g1_ref, dis_ref, b1_ref, g2_ref):
    dis = dis_ref[...]
    pre = (s1a_ref[...] + s1b_ref[...] + g1_ref[...]) * dis + b1_ref[...]
    h1 = jnp.where(pre >= 0, pre, 0.1 * pre)
    g2_ref[...] = h1 * dis


_tc_mid = pl.pallas_call(
    _tc_mid_body,
    grid=(_GRID,),
    in_specs=[
        pl.BlockSpec((_B, 3), lambda i: (i, 0)),
        pl.BlockSpec((_B, 3), lambda i: (i, 0)),
        pl.BlockSpec((_B, 3), lambda i: (i, 0)),
        pl.BlockSpec((_B, 1), lambda i: (i, 0)),
        pl.BlockSpec((1, 3), lambda i: (0, 0)),
    ],
    out_specs=pl.BlockSpec((_B, 3), lambda i: (i, 0)),
    out_shape=jax.ShapeDtypeStruct((N_NODES, 3), jnp.float32),
)


def _tc_lin_body(s2a_ref, s2b_ref, g2_ref, dis_ref, w2_ref, b2_ref, h2_ref):
    p = (s2a_ref[...] + s2b_ref[...] + g2_ref[...]) * dis_ref[...]
    h2 = lax.dot_general(p, w2_ref[...], (((1,), (1,)), ((), ())),
                         preferred_element_type=jnp.float32)
    h2_ref[...] = h2 + b2_ref[...]


_tc_lin = pl.pallas_call(
    _tc_lin_body,
    grid=(_GRID,),
    in_specs=[
        pl.BlockSpec((_B, 3), lambda i: (i, 0)),
        pl.BlockSpec((_B, 3), lambda i: (i, 0)),
        pl.BlockSpec((_B, 3), lambda i: (i, 0)),
        pl.BlockSpec((_B, 1), lambda i: (i, 0)),
        pl.BlockSpec((16, 3), lambda i: (0, 0)),
        pl.BlockSpec((1, 16), lambda i: (0, 0)),
    ],
    out_specs=pl.BlockSpec((_B, 16), lambda i: (i, 0)),
    out_shape=jax.ShapeDtypeStruct((N_NODES, 16), jnp.float32),
)


def _tc_final_body(h2_ref, w3_ref, b3_ref, out_ref):
    out = lax.dot_general(h2_ref[...], w3_ref[...], (((1,), (1,)), ((), ())),
                          preferred_element_type=jnp.float32)
    out_ref[...] = out + b3_ref[...]


_tc_final = pl.pallas_call(
    _tc_final_body,
    out_shape=jax.ShapeDtypeStruct((50, 128), jnp.float32),
)


def kernel(nodes, edges, W1, b1, W2, b2, W3, b3):
    src = edges[0, 0].astype(jnp.int32)
    dst = edges[0, 1].astype(jnp.int32)
    ones = jnp.ones((CHUNK, 1), jnp.float32)
    zeros1 = jnp.zeros((N_PAD, 1), jnp.float32)
    zeros3 = jnp.zeros((N_PAD, 3), jnp.float32)

    cnt = _sc_count(dst, ones, zeros1)
    g1, dis = _tc_prep(cnt[:, :N_NODES, 0].T, nodes, W1)
    s1 = _sc_scatter(g1, src, dst, zeros3)
    g2 = _tc_mid(s1[0, :N_NODES], s1[1, :N_NODES], g1, dis,
                 b1.reshape(1, 3))
    s2 = _sc_scatter(g2, src, dst, zeros3)
    h2 = _tc_lin(s2[0, :N_NODES], s2[1, :N_NODES], g2, dis, W2,
                 b2.reshape(1, 16))
    out = _tc_final(h2.reshape(50, 16000), W3, b3.reshape(1, 128))
    return out


# speedup vs baseline: 61.4424x; 1.5334x over previous
import functools

import jax
import jax.numpy as jnp
from jax import lax
from jax.experimental import pallas as pl
from jax.experimental.pallas import tpu as pltpu
import jax.experimental.pallas.tpu_sc as plsc

N_NODES = 50000
N_EDGES = 1600000
NC = 2
NS = 16
NW = NC * NS
N_PAD = 50048
R_TILE = N_PAD // NS
CHUNK = 2000
PER_W = N_EDGES // NW

_mesh = plsc.VectorSubcoreMesh(core_axis_name="c", subcore_axis_name="s")
_sc_params = pltpu.CompilerParams(use_tc_tiling_on_sc=False)


@functools.partial(
    pl.kernel,
    out_type=jax.ShapeDtypeStruct((NC, N_PAD, 1), jnp.float32),
    mesh=_mesh,
    compiler_params=_sc_params,
    scratch_types=[
        pltpu.VMEM((CHUNK,), jnp.int32),
        pltpu.VMEM((CHUNK, 1), jnp.float32),
        pltpu.VMEM((R_TILE, 1), jnp.float32),
        pltpu.VMEM_SHARED((N_PAD, 1), jnp.float32),
        pltpu.SemaphoreType.DMA,
    ],
)
def _sc_count(dst_hbm, ones_hbm, zeros_hbm, out_hbm,
              dst_v, ones_v, buf_v, acc_sh, sem):
    cid = lax.axis_index("c")
    sid = lax.axis_index("s")
    wid = cid * NS + sid
    pltpu.sync_copy(zeros_hbm, buf_v)
    pltpu.sync_copy(buf_v, acc_sh.at[pl.ds(sid * R_TILE, R_TILE)])
    pltpu.sync_copy(ones_hbm, ones_v)
    plsc.subcore_barrier()
    base = wid * PER_W

    def body(i, carry):
        off = base + i * CHUNK
        pltpu.sync_copy(dst_hbm.at[pl.ds(off, CHUNK)], dst_v)
        pltpu.sync_copy(ones_v, acc_sh.at[dst_v], add=True)
        return carry

    lax.fori_loop(0, PER_W // CHUNK, body, 0)
    plsc.subcore_barrier()
    pltpu.sync_copy(acc_sh.at[pl.ds(sid * R_TILE, R_TILE)], buf_v)
    pltpu.sync_copy(buf_v, out_hbm.at[cid, pl.ds(sid * R_TILE, R_TILE)])


@functools.partial(
    pl.kernel,
    out_type=jax.ShapeDtypeStruct((NC, N_PAD, 3), jnp.float32),
    mesh=_mesh,
    compiler_params=_sc_params,
    scratch_types=[
        pltpu.VMEM((CHUNK,), jnp.int32),
        pltpu.VMEM((CHUNK,), jnp.int32),
        pltpu.VMEM((CHUNK, 3), jnp.float32),
        pltpu.VMEM((R_TILE, 3), jnp.float32),
        pltpu.VMEM_SHARED((N_PAD, 3), jnp.float32),
        pltpu.SemaphoreType.DMA,
    ],
)
def _sc_scatter(g_hbm, src_hbm, dst_hbm, zeros_hbm, out_hbm,
                src_v, dst_v, rows_v, buf_v, acc_sh, sem):
    cid = lax.axis_index("c")
    sid = lax.axis_index("s")
    wid = cid * NS + sid
    pltpu.sync_copy(zeros_hbm, buf_v)
    pltpu.sync_copy(buf_v, acc_sh.at[pl.ds(sid * R_TILE, R_TILE)])
    plsc.subcore_barrier()
    base = wid * PER_W

    def body(i, carry):
        off = base + i * CHUNK
        pltpu.sync_copy(src_hbm.at[pl.ds(off, CHUNK)], src_v)
        pltpu.async_copy(g_hbm.at[src_v], rows_v, sem).wait()
        pltpu.sync_copy(dst_hbm.at[pl.ds(off, CHUNK)], dst_v)
        pltpu.sync_copy(rows_v, acc_sh.at[dst_v], add=True)
        return carry

    lax.fori_loop(0, PER_W // CHUNK, body, 0)
    plsc.subcore_barrier()
    pltpu.sync_copy(acc_sh.at[pl.ds(sid * R_TILE, R_TILE)], buf_v)
    pltpu.sync_copy(buf_v, out_hbm.at[cid, pl.ds(sid * R_TILE, R_TILE)])


_B = 1000
_GRID = N_NODES // _B


def _tc_prep_body(cnt0_ref, cnt1_ref, nodes_ref, w1_ref, g1_ref, dis_ref):
    deg = cnt0_ref[0, :, 0] + cnt1_ref[0, :, 0] + 1.0
    dis = lax.rsqrt(deg)[:, None]
    h = lax.dot_general(nodes_ref[...], w1_ref[...],
                        (((1,), (1,)), ((), ())),
                        preferred_element_type=jnp.float32)
    g1_ref[...] = h * dis
    dis_ref[...] = dis


_tc_prep = pl.pallas_call(
    _tc_prep_body,
    grid=(_GRID,),
    in_specs=[
        pl.BlockSpec((1, _B, 1), lambda i: (0, i, 0)),
        pl.BlockSpec((1, _B, 1), lambda i: (1, i, 0)),
        pl.BlockSpec((_B, 3), lambda i: (i, 0)),
        pl.BlockSpec((3, 3), lambda i: (0, 0)),
    ],
    out_specs=[
        pl.BlockSpec((_B, 3), lambda i: (i, 0)),
        pl.BlockSpec((_B, 1), lambda i: (i, 0)),
    ],
    out_shape=[
        jax.ShapeDtypeStruct((N_NODES, 3), jnp.float32),
        jax.ShapeDtypeStruct((N_NODES, 1), jnp.float32),
    ],
)


def _tc_mid_body(s1a_ref, s1b_ref, g1_ref, dis_ref, b1_ref, g2_ref):
    dis = dis_ref[...]
    pre = (s1a_ref[0] + s1b_ref[0] + g1_ref[...]) * dis + b1_ref[...]
    h1 = jnp.where(pre >= 0, pre, 0.1 * pre)
    g2_ref[...] = h1 * dis


_tc_mid = pl.pallas_call(
    _tc_mid_body,
    grid=(_GRID,),
    in_specs=[
        pl.BlockSpec((1, _B, 3), lambda i: (0, i, 0)),
        pl.BlockSpec((1, _B, 3), lambda i: (1, i, 0)),
        pl.BlockSpec((_B, 3), lambda i: (i, 0)),
        pl.BlockSpec((_B, 1), lambda i: (i, 0)),
        pl.BlockSpec((1, 3), lambda i: (0, 0)),
    ],
    out_specs=pl.BlockSpec((_B, 3), lambda i: (i, 0)),
    out_shape=jax.ShapeDtypeStruct((N_NODES, 3), jnp.float32),
)


def _tc_lin_body(s2a_ref, s2b_ref, g2_ref, dis_ref, w2_ref, b2_ref, h2_ref):
    p = (s2a_ref[0] + s2b_ref[0] + g2_ref[...]) * dis_ref[...]
    h2 = lax.dot_general(p, w2_ref[...], (((1,), (1,)), ((), ())),
                         preferred_element_type=jnp.float32)
    h2_ref[...] = h2 + b2_ref[...]


_tc_lin = pl.pallas_call(
    _tc_lin_body,
    grid=(_GRID,),
    in_specs=[
        pl.BlockSpec((1, _B, 3), lambda i: (0, i, 0)),
        pl.BlockSpec((1, _B, 3), lambda i: (1, i, 0)),
        pl.BlockSpec((_B, 3), lambda i: (i, 0)),
        pl.BlockSpec((_B, 1), lambda i: (i, 0)),
        pl.BlockSpec((16, 3), lambda i: (0, 0)),
        pl.BlockSpec((1, 16), lambda i: (0, 0)),
    ],
    out_specs=pl.BlockSpec((_B, 16), lambda i: (i, 0)),
    out_shape=jax.ShapeDtypeStruct((N_NODES, 16), jnp.float32),
)


def _tc_final_body(h2_ref, w3_ref, b3_ref, out_ref):
    out = lax.dot_general(h2_ref[...], w3_ref[...], (((1,), (1,)), ((), ())),
                          preferred_element_type=jnp.float32)
    out_ref[...] = out + b3_ref[...]


_tc_final = pl.pallas_call(
    _tc_final_body,
    out_shape=jax.ShapeDtypeStruct((50, 128), jnp.float32),
)


def kernel(nodes, edges, W1, b1, W2, b2, W3, b3):
    src = edges[0, 0].astype(jnp.int32)
    dst = edges[0, 1].astype(jnp.int32)
    ones = jnp.ones((CHUNK, 1), jnp.float32)
    zeros1 = jnp.zeros((R_TILE, 1), jnp.float32)
    zeros3 = jnp.zeros((R_TILE, 3), jnp.float32)

    cnt = _sc_count(dst, ones, zeros1)
    g1, dis = _tc_prep(cnt, cnt, nodes, W1)
    s1 = _sc_scatter(g1, src, dst, zeros3)
    g2 = _tc_mid(s1, s1, g1, dis, b1.reshape(1, 3))
    s2 = _sc_scatter(g2, src, dst, zeros3)
    h2 = _tc_lin(s2, s2, g2, dis, W2, b2.reshape(1, 16))
    out = _tc_final(h2.reshape(50, 16000), W3, b3.reshape(1, 128))
    return out
